# baseline (speedup 1.0000x reference)
import jax
import jax.numpy as jnp
from jax import lax
from jax.experimental import pallas as pl
from jax.experimental.pallas import tpu as pltpu

N_SP = 100
N_SP_PAD = 128
N_EXP = 16
N_RTD = 8
NSPLIT = 4
TILE = 1024


def _body(z0, z1, z2, z3, f0, f1, f2, f3, embt_ref, wr_ref, wall_ref, b_ref,
          o0, o1, o2, o3, wcomb_ref, bcomb_ref):
    @pl.when(pl.program_id(0) == 0)
    def _():
        embt = embt_ref[...]
        u = embt * (1.0 / (1.0 + jnp.exp(-embt)))
        logits = jnp.dot(wr_ref[...], u,
                         preferred_element_type=jnp.float32)
        row = lax.broadcasted_iota(jnp.int32, (N_EXP, N_SP_PAD), 0)
        valid = row < N_RTD
        lm = jnp.max(jnp.where(valid, logits, jnp.float32(-1e30)),
                     axis=0, keepdims=True)
        ex = jnp.where(valid, jnp.exp(logits - lm), 0.0)
        s = ex / jnp.sum(ex, axis=0, keepdims=True)
        m1 = jnp.max(s, axis=0, keepdims=True)
        i1 = jnp.min(jnp.where((s == m1) & valid, row, N_EXP),
                     axis=0, keepdims=True)
        msk2 = valid & (row != i1)
        sm = jnp.where(msk2, s, -1.0)
        m2 = jnp.max(sm, axis=0, keepdims=True)
        i2 = jnp.min(jnp.where(sm == m2, row, N_EXP), axis=0, keepdims=True)
        keep = (row == i1) | (row == i2)
        coef_t = jnp.where(valid, jnp.where(keep, s, 0.0), 1.0)
        wcomb_ref[...] = jnp.dot(wall_ref[...], coef_t,
                                 preferred_element_type=jnp.float32)
        bcomb_ref[...] = jnp.dot(b_ref[...], coef_t,
                                 preferred_element_type=jnp.float32)

    wcomb = wcomb_ref[...]
    bc = bcomb_ref[0:1, :]
    ones_col = jnp.ones((N_SP_PAD, 1), jnp.float32)
    for f_ref, z_ref, o_ref in ((f0, z0, o0), (f1, z1, o1),
                                (f2, z2, o2), (f3, z3, o3)):
        o = jnp.dot(f_ref[...], wcomb,
                    preferred_element_type=jnp.float32)
        p = o + bc
        z = z_ref[...]
        sp = lax.broadcasted_iota(jnp.int32, (z.shape[0], N_SP_PAD), 1)
        q = jnp.where(z == sp, p, 0.0)
        o_ref[...] = jnp.dot(q, ones_col,
                             preferred_element_type=jnp.float32)


def kernel(features, species_idx, emb, W_router, W_experts, b_experts):
    n, in_f = features.shape
    n_species, embd = emb.shape
    wall = W_experts[:, 0, :].T
    wr = jnp.zeros((N_EXP, embd), jnp.float32).at[:N_RTD].set(W_router)
    embt = jnp.zeros((embd, N_SP_PAD), jnp.float32).at[:, :n_species].set(emb.T)
    b_rep = jnp.broadcast_to(b_experts.reshape(1, N_EXP), (8, N_EXP))
    z2d = species_idx.astype(jnp.int32).reshape(n, 1)

    npart = n // NSPLIT
    g = npart // TILE

    def zmap(k):
        return lambda i: (i + k * g, 0)

    outs = pl.pallas_call(
        _body,
        grid=(g,),
        in_specs=[
            pl.BlockSpec((TILE, 1), zmap(0)),
            pl.BlockSpec((TILE, 1), zmap(1)),
            pl.BlockSpec((TILE, 1), zmap(2)),
            pl.BlockSpec((TILE, 1), zmap(3)),
            pl.BlockSpec((TILE, in_f), zmap(0)),
            pl.BlockSpec((TILE, in_f), zmap(1)),
            pl.BlockSpec((TILE, in_f), zmap(2)),
            pl.BlockSpec((TILE, in_f), zmap(3)),
            pl.BlockSpec((embd, N_SP_PAD), lambda i: (0, 0)),
            pl.BlockSpec((N_EXP, embd), lambda i: (0, 0)),
            pl.BlockSpec((in_f, N_EXP), lambda i: (0, 0)),
            pl.BlockSpec((8, N_EXP), lambda i: (0, 0)),
        ],
        out_specs=[pl.BlockSpec((TILE, 1), lambda i: (i, 0))] * NSPLIT,
        out_shape=[jax.ShapeDtypeStruct((npart, 1), jnp.float32)] * NSPLIT,
        scratch_shapes=[
            pltpu.VMEM((in_f, N_SP_PAD), jnp.float32),
            pltpu.VMEM((8, N_SP_PAD), jnp.float32),
        ],
    )(z2d, z2d, z2d, z2d, features, features, features, features,
      embt, wr, wall, b_rep)
    return jnp.concatenate(outs, axis=0)


# speedup vs baseline: 1.0474x; 1.0474x over previous
import jax
import jax.numpy as jnp
from jax import lax
from jax.experimental import pallas as pl
from jax.experimental.pallas import tpu as pltpu

N_SP = 100
N_SP_PAD = 128
N_EXP = 16
N_RTD = 8
NSPLIT = 4
TILE = 1024


def _body(z0, z1, z2, z3, f0, f1, f2, f3, embt_ref, wr_ref, wall_ref, b_ref,
          o0, o1, o2, o3, wcomb_ref, bcomb_ref):
    @pl.when(pl.program_id(0) == 0)
    def _():
        embt = embt_ref[...]
        u = embt * (1.0 / (1.0 + jnp.exp(-embt)))
        logits = jnp.dot(wr_ref[...], u,
                         preferred_element_type=jnp.float32)
        row = lax.broadcasted_iota(jnp.int32, (N_EXP, N_SP_PAD), 0)
        valid = row < N_RTD
        lm = jnp.max(jnp.where(valid, logits, jnp.float32(-1e30)),
                     axis=0, keepdims=True)
        ex = jnp.where(valid, jnp.exp(logits - lm), 0.0)
        s = ex / jnp.sum(ex, axis=0, keepdims=True)
        m1 = jnp.max(s, axis=0, keepdims=True)
        i1 = jnp.min(jnp.where((s == m1) & valid, row, N_EXP),
                     axis=0, keepdims=True)
        msk2 = valid & (row != i1)
        sm = jnp.where(msk2, s, -1.0)
        m2 = jnp.max(sm, axis=0, keepdims=True)
        i2 = jnp.min(jnp.where(sm == m2, row, N_EXP), axis=0, keepdims=True)
        keep = (row == i1) | (row == i2)
        coef_t = jnp.where(valid, jnp.where(keep, s, 0.0), 1.0)
        wcomb_ref[...] = jnp.dot(wall_ref[...], coef_t,
                                 preferred_element_type=jnp.float32)
        bcomb_ref[...] = jnp.dot(b_ref[...], coef_t,
                                 preferred_element_type=jnp.float32)

    wcomb = wcomb_ref[...]
    bc = bcomb_ref[0:1, :]
    for f_ref, z_ref, o_ref in ((f0, z0, o0), (f1, z1, o1),
                                (f2, z2, o2), (f3, z3, o3)):
        o = jnp.dot(f_ref[...], wcomb,
                    preferred_element_type=jnp.float32)
        p = o + bc
        z = z_ref[...]
        sp = lax.broadcasted_iota(jnp.int32, (z.shape[0], N_SP_PAD), 1)
        sel = (z == sp).astype(jnp.float32)
        o_ref[...] = jnp.sum(sel * p, axis=1, keepdims=True)


def kernel(features, species_idx, emb, W_router, W_experts, b_experts):
    n, in_f = features.shape
    n_species, embd = emb.shape
    wall = W_experts[:, 0, :].T
    wr = jnp.zeros((N_EXP, embd), jnp.float32).at[:N_RTD].set(W_router)
    embt = jnp.zeros((embd, N_SP_PAD), jnp.float32).at[:, :n_species].set(emb.T)
    b_rep = jnp.broadcast_to(b_experts.reshape(1, N_EXP), (8, N_EXP))
    z2d = species_idx.astype(jnp.int32).reshape(n, 1)

    npart = n // NSPLIT
    g = npart // TILE

    def zmap(k):
        return lambda i: (i + k * g, 0)

    outs = pl.pallas_call(
        _body,
        grid=(g,),
        in_specs=[
            pl.BlockSpec((TILE, 1), zmap(0)),
            pl.BlockSpec((TILE, 1), zmap(1)),
            pl.BlockSpec((TILE, 1), zmap(2)),
            pl.BlockSpec((TILE, 1), zmap(3)),
            pl.BlockSpec((TILE, in_f), zmap(0)),
            pl.BlockSpec((TILE, in_f), zmap(1)),
            pl.BlockSpec((TILE, in_f), zmap(2)),
            pl.BlockSpec((TILE, in_f), zmap(3)),
            pl.BlockSpec((embd, N_SP_PAD), lambda i: (0, 0)),
            pl.BlockSpec((N_EXP, embd), lambda i: (0, 0)),
            pl.BlockSpec((in_f, N_EXP), lambda i: (0, 0)),
            pl.BlockSpec((8, N_EXP), lambda i: (0, 0)),
        ],
        out_specs=[pl.BlockSpec((TILE, 1), lambda i: (i, 0))] * NSPLIT,
        out_shape=[jax.ShapeDtypeStruct((npart, 1), jnp.float32)] * NSPLIT,
        scratch_shapes=[
            pltpu.VMEM((in_f, N_SP_PAD), jnp.float32),
            pltpu.VMEM((8, N_SP_PAD), jnp.float32),
        ],
    )(z2d, z2d, z2d, z2d, features, features, features, features,
      embt, wr, wall, b_rep)
    return jnp.concatenate(outs, axis=0)
